# Initial kernel scaffold; baseline (speedup 1.0000x reference)
import jax
import jax.numpy as jnp
from jax.experimental import pallas as pl


def kernel(x, Wc, bc, W1, b1, W2, b2, W3, b3):
    raise NotImplementedError("write your pallas kernel here")


# speedup vs baseline: 19.9132x; 19.9132x over previous
import jax
import jax.numpy as jnp
from jax import lax
from jax.experimental import pallas as pl
from jax.experimental.pallas import tpu as pltpu

_B, _F, _N = 32, 64, 32768
_J, _R = 32, 16


def _minmax_body(x_ref, wc_ref, bc_ref, feat_ref, vals_ref):
    xb = x_ref[0]
    wc = wc_ref[...]
    emb = lax.dot_general(
        wc, xb, (((1,), (0,)), ((), ())),
        preferred_element_type=jnp.float32,
        precision=lax.Precision.HIGHEST,
    ) + bc_ref[...]
    vals_ref[...] = emb

    lane = lax.broadcasted_iota(jnp.int32, (_J, 2 * _R), 1)
    neg = jnp.float32(-jnp.inf)
    pos = jnp.float32(jnp.inf)

    def body(_, carry):
        feat, t_hi, t_lo, p_hi, p_lo = carry
        v = vals_ref[...]
        m_hi = jnp.max(jnp.where(v < t_hi, v, neg), axis=1, keepdims=True)
        c_hi = jnp.sum((v == m_hi).astype(jnp.int32), axis=1, keepdims=True)
        m_lo = jnp.min(jnp.where(v > t_lo, v, pos), axis=1, keepdims=True)
        c_lo = jnp.sum((v == m_lo).astype(jnp.int32), axis=1, keepdims=True)
        fill_hi = (lane >= p_hi) & (lane < p_hi + c_hi) & (lane < _R)
        fill_lo = (lane <= p_lo) & (lane > p_lo - c_lo) & (lane >= _R)
        feat = jnp.where(fill_hi, m_hi, feat)
        feat = jnp.where(fill_lo, m_lo, feat)
        return (feat, m_hi, m_lo, p_hi + c_hi, p_lo - c_lo)

    init = (
        jnp.zeros((_J, 2 * _R), jnp.float32),
        jnp.full((_J, 1), pos),
        jnp.full((_J, 1), neg),
        jnp.zeros((_J, 1), jnp.int32),
        jnp.full((_J, 1), 2 * _R - 1, jnp.int32),
    )
    feat = lax.fori_loop(0, _R, body, init)[0]
    feat_ref[0] = feat


def _mlp_body(f_ref, w1_ref, b1_ref, w2_ref, b2_ref, w3_ref, b3_ref, out_ref):
    def dot(a, b):
        return lax.dot_general(
            a, b, (((1,), (0,)), ((), ())),
            preferred_element_type=jnp.float32,
            precision=lax.Precision.HIGHEST,
        )

    h = dot(f_ref[...], w1_ref[...]) + b1_ref[...]
    h = 1.0 / (1.0 + jnp.exp(-h))
    h = dot(h, w2_ref[...]) + b2_ref[...]
    h = 1.0 / (1.0 + jnp.exp(-h))
    out_ref[...] = dot(h, w3_ref[...]) + b3_ref[...]


def kernel(x, Wc, bc, W1, b1, W2, b2, W3, b3):
    feats = pl.pallas_call(
        _minmax_body,
        grid=(_B,),
        in_specs=[
            pl.BlockSpec((1, _F, _N), lambda b: (b, 0, 0)),
            pl.BlockSpec((_J, _F), lambda b: (0, 0)),
            pl.BlockSpec((_J, 1), lambda b: (0, 0)),
        ],
        out_specs=pl.BlockSpec((1, _J, 2 * _R), lambda b: (b, 0, 0)),
        out_shape=jax.ShapeDtypeStruct((_B, _J, 2 * _R), jnp.float32),
        scratch_shapes=[pltpu.VMEM((_J, _N), jnp.float32)],
    )(x, Wc, bc.reshape(_J, 1))

    flat = feats.reshape(_B, _J * 2 * _R)
    logits = pl.pallas_call(
        _mlp_body,
        out_shape=jax.ShapeDtypeStruct((_B, 2), jnp.float32),
    )(flat, W1, b1.reshape(1, -1), W2, b2.reshape(1, -1),
      W3, b3.reshape(1, -1))
    return logits


# speedup vs baseline: 41.6635x; 2.0923x over previous
import functools

import jax
import jax.numpy as jnp
from jax import lax
from jax.experimental import pallas as pl
from jax.experimental.pallas import tpu as pltpu
from jax.experimental.pallas import tpu_sc as plsc

_B, _F, _N = 32, 64, 32768
_J, _R = 32, 16
_BLK = 128
_NBLK = _N // _BLK
_NROW = _B * _J
_L = 16
_NC, _NS = 2, 16
_NW = _NC * _NS
_RPW = _NROW // _NW
_SROW = _RPW * _NBLK


def _emb_stats_body(x_ref, wc_ref, bc_ref, emb_ref, bmax_ref, bmin_ref):
    xb = x_ref[0]
    emb = lax.dot_general(
        wc_ref[...], xb, (((1,), (0,)), ((), ())),
        preferred_element_type=jnp.float32,
        precision=lax.Precision.HIGHEST,
    ) + bc_ref[...]
    emb_ref[0] = emb
    r = emb.reshape(_J, _NBLK, _BLK)
    bmax_ref[0] = jnp.max(r, axis=2)
    bmin_ref[0] = jnp.min(r, axis=2)


def _sc_body(emb_hbm, bmax_hbm, bmin_hbm, out_hbm,
             stats_v, idx_v, cand_v, buf_v, feat_v, sem):
    wid = lax.axis_index("s") * _NC + lax.axis_index("c")
    base = wid * _RPW
    pltpu.sync_copy(bmax_hbm.at[pl.ds(base * _NBLK, _SROW)],
                    stats_v.at[pl.ds(0, _SROW)])
    pltpu.sync_copy(bmin_hbm.at[pl.ds(base * _NBLK, _SROW)],
                    stats_v.at[pl.ds(_SROW, _SROW)])

    neg_inf = jnp.float32(-jnp.inf)
    lane = lax.iota(jnp.int32, _L)

    def side(rloc, stats_off, negate):
        runv = jnp.full((_L,), neg_inf)
        runi = jnp.zeros((_L,), jnp.int32)
        for c in range(_NBLK // _L):
            v = stats_v[pl.ds(stats_off + rloc * _NBLK + c * _L, _L)]
            if negate:
                v = -v
            vd, idd = plsc.sort_key_val(v, lane + c * _L, descending=True)
            keep = runv >= vd
            mv = jnp.where(keep, runv, vd)
            mi = jnp.where(keep, runi, idd)
            runv, runi = plsc.sort_key_val(mv, mi)
        t = jnp.min(runv)
        idx_v[...] = runi + (base + rloc) * _NBLK
        pltpu.async_copy(emb_hbm.at[idx_v], cand_v, sem).wait()
        for i in range(_L * _BLK // _L + 1):
            buf_v[pl.ds(i * _L, _L)] = jnp.full((_L,), neg_inf)
        ptr = jnp.int32(0)
        for j in range(_L):
            for k in range(_BLK // _L):
                cvec = cand_v[j, pl.ds(k * _L, _L)]
                if negate:
                    cvec = -cvec
                m = cvec >= t
                plsc.store_compressed(buf_v.at[pl.ds(ptr, _L)], cvec, mask=m)
                ptr = ptr + jnp.sum(m.astype(jnp.int32))
        def fbody(i, rv):
            cd = plsc.sort_key_val(buf_v[pl.ds(i * _L, _L)], lane,
                                   descending=True)[0]
            return lax.sort(jnp.where(rv >= cd, rv, cd))
        nv = (ptr + _L - 1) // _L
        return lax.fori_loop(0, nv, fbody, jnp.full((_L,), neg_inf))

    def row_body(rloc, carry):
        top = side(rloc, 0, False)
        feat_v[pl.ds(rloc * 2 * _R, _R)] = lax.rev(top, (0,))
        bot = side(rloc, _SROW, True)
        feat_v[pl.ds(rloc * 2 * _R + _R, _R)] = -bot
        return carry

    lax.fori_loop(0, _RPW, row_body, 0)
    pltpu.sync_copy(feat_v, out_hbm.at[pl.ds(base * 2 * _R, _RPW * 2 * _R)])


_sc_select = functools.partial(
    pl.kernel,
    out_type=jax.ShapeDtypeStruct((_NROW * 2 * _R,), jnp.float32),
    mesh=plsc.VectorSubcoreMesh(core_axis_name="c", subcore_axis_name="s",
                                num_cores=_NC, num_subcores=_NS),
    compiler_params=pltpu.CompilerParams(needs_layout_passes=False),
    scratch_types=[
        pltpu.VMEM((2 * _SROW,), jnp.float32),
        pltpu.VMEM((_L,), jnp.int32),
        pltpu.VMEM((_L, _BLK), jnp.float32),
        pltpu.VMEM((_L * _BLK + _L,), jnp.float32),
        pltpu.VMEM((_RPW * 2 * _R,), jnp.float32),
        pltpu.SemaphoreType.DMA,
    ],
)(_sc_body)


def _mlp_body(f_ref, w1_ref, b1_ref, w2_ref, b2_ref, w3_ref, b3_ref, out_ref):
    def dot(a, b):
        return lax.dot_general(
            a, b, (((1,), (0,)), ((), ())),
            preferred_element_type=jnp.float32,
            precision=lax.Precision.HIGHEST,
        )

    h = dot(f_ref[...], w1_ref[...]) + b1_ref[...]
    h = 1.0 / (1.0 + jnp.exp(-h))
    h = dot(h, w2_ref[...]) + b2_ref[...]
    h = 1.0 / (1.0 + jnp.exp(-h))
    out_ref[...] = dot(h, w3_ref[...]) + b3_ref[...]


def kernel(x, Wc, bc, W1, b1, W2, b2, W3, b3):
    emb, bmax, bmin = pl.pallas_call(
        _emb_stats_body,
        grid=(_B,),
        in_specs=[
            pl.BlockSpec((1, _F, _N), lambda b: (b, 0, 0)),
            pl.BlockSpec((_J, _F), lambda b: (0, 0)),
            pl.BlockSpec((_J, 1), lambda b: (0, 0)),
        ],
        out_specs=[
            pl.BlockSpec((1, _J, _N), lambda b: (b, 0, 0)),
            pl.BlockSpec((1, _J, _NBLK), lambda b: (b, 0, 0)),
            pl.BlockSpec((1, _J, _NBLK), lambda b: (b, 0, 0)),
        ],
        out_shape=[
            jax.ShapeDtypeStruct((_B, _J, _N), jnp.float32),
            jax.ShapeDtypeStruct((_B, _J, _NBLK), jnp.float32),
            jax.ShapeDtypeStruct((_B, _J, _NBLK), jnp.float32),
        ],
    )(x, Wc, bc.reshape(_J, 1))

    feats = _sc_select(emb.reshape(_NROW * _NBLK, _BLK),
                       bmax.reshape(-1), bmin.reshape(-1))

    flat = feats.reshape(_B, _J * 2 * _R)
    logits = pl.pallas_call(
        _mlp_body,
        out_shape=jax.ShapeDtypeStruct((_B, 2), jnp.float32),
    )(flat, W1, b1.reshape(1, -1), W2, b2.reshape(1, -1),
      W3, b3.reshape(1, -1))
    return logits


# speedup vs baseline: 79.5577x; 1.9095x over previous
import functools

import jax
import jax.numpy as jnp
from jax import lax
from jax.experimental import pallas as pl
from jax.experimental.pallas import tpu as pltpu
from jax.experimental.pallas import tpu_sc as plsc

_B, _F, _N = 32, 64, 32768
_J, _R = 32, 16
_BLK = 128
_NBLK = _N // _BLK
_NROW = _B * _J
_L = 16
_NC, _NS = 2, 16
_NW = _NC * _NS
_RPW = _NROW // _NW
_SROW = _RPW * _NBLK


def _emb_stats_body(x_ref, wc_ref, bc_ref, emb_ref, bmax_ref, bmin_ref):
    xb = x_ref[0]
    emb = lax.dot_general(
        wc_ref[...], xb, (((1,), (0,)), ((), ())),
        preferred_element_type=jnp.float32,
        precision=lax.Precision.HIGHEST,
    ) + bc_ref[...]
    r = emb.reshape(_J, _NBLK, _BLK)
    emb_ref[0] = r
    bmax_ref[0] = jnp.max(r, axis=2)
    bmin_ref[0] = jnp.min(r, axis=2)


def _sc_body(emb_hbm, bmax_hbm, bmin_hbm, out_hbm,
             stats_v, it0, ib0, it1, ib1, ct0, cb0, ct1, cb1,
             buf_v, feat_v, st0, sb0, st1, sb1):
    wid = lax.axis_index("s") * _NC + lax.axis_index("c")
    base = wid * _RPW
    pltpu.sync_copy(bmax_hbm.at[pl.ds(base * _NBLK, _SROW)],
                    stats_v.at[pl.ds(0, _SROW)])
    pltpu.sync_copy(bmin_hbm.at[pl.ds(base * _NBLK, _SROW)],
                    stats_v.at[pl.ds(_SROW, _SROW)])

    neg_inf = jnp.float32(-jnp.inf)
    lane = lax.iota(jnp.int32, _L)

    def select2(rloc):
        runvT = jnp.full((_L,), neg_inf)
        runiT = jnp.zeros((_L,), jnp.int32)
        runvB = jnp.full((_L,), neg_inf)
        runiB = jnp.zeros((_L,), jnp.int32)
        for c in range(_NBLK // _L):
            ids = lane + c * _L
            vT = stats_v[pl.ds(rloc * _NBLK + c * _L, _L)]
            vB = -stats_v[pl.ds(_SROW + rloc * _NBLK + c * _L, _L)]
            vdT, idT = plsc.sort_key_val(vT, ids, descending=True)
            vdB, idB = plsc.sort_key_val(vB, ids, descending=True)
            kT = runvT >= vdT
            kB = runvB >= vdB
            runvT, runiT = plsc.sort_key_val(jnp.where(kT, runvT, vdT),
                                             jnp.where(kT, runiT, idT))
            runvB, runiB = plsc.sort_key_val(jnp.where(kB, runvB, vdB),
                                             jnp.where(kB, runiB, idB))
        return runiT, jnp.min(runvT), runiB, jnp.min(runvB)

    def sel_fire(rloc, idxT, candT, semT, idxB, candB, semB):
        rl = jnp.where(rloc >= _RPW, 0, rloc)
        riT, tT, riB, tB = select2(rl)
        gbase = (base + rl) * _NBLK
        idxT[...] = riT + gbase
        pltpu.async_copy(emb_hbm.at[idxT], candT, semT)
        idxB[...] = riB + gbase
        pltpu.async_copy(emb_hbm.at[idxB], candB, semB)
        return tT, tB

    def process(cand, t, negate):
        def jbody(j, ptr):
            for k in range(_BLK // _L):
                cvec = cand[j, pl.ds(k * _L, _L)]
                if negate:
                    cvec = -cvec
                m = cvec >= t
                plsc.store_compressed(buf_v.at[pl.ds(ptr, _L)], cvec, mask=m)
                ptr = ptr + plsc.all_reduce_population_count(m)[0]
            return ptr
        ptr = lax.fori_loop(0, _L, jbody, jnp.int32(0))
        buf_v[pl.ds(ptr, _L)] = jnp.full((_L,), neg_inf)
        def fbody(i, rv):
            cd = plsc.sort_key_val(buf_v[pl.ds(i * _L, _L)], lane,
                                   descending=True)[0]
            return lax.sort(jnp.where(rv >= cd, rv, cd))
        nv = (ptr + _L - 1) // _L
        return lax.fori_loop(0, nv, fbody, jnp.full((_L,), neg_inf))

    def finish(rloc, idxT, candT, semT, idxB, candB, semB, tT, tB):
        pltpu.make_async_copy(emb_hbm.at[idxT], candT, semT).wait()
        pltpu.make_async_copy(emb_hbm.at[idxB], candB, semB).wait()
        top = process(candT, tT, False)
        feat_v[pl.ds(rloc * 2 * _R, _R)] = lax.rev(top, (0,))
        bot = process(candB, tB, True)
        feat_v[pl.ds(rloc * 2 * _R + _R, _R)] = -bot

    t0 = sel_fire(jnp.int32(0), it0, ct0, st0, ib0, cb0, sb0)

    def pair_body(i, carry):
        tT0, tB0 = carry
        r0 = 2 * i
        tT1, tB1 = sel_fire(r0 + 1, it1, ct1, st1, ib1, cb1, sb1)
        finish(r0, it0, ct0, st0, ib0, cb0, sb0, tT0, tB0)
        nxt = sel_fire(r0 + 2, it0, ct0, st0, ib0, cb0, sb0)
        finish(r0 + 1, it1, ct1, st1, ib1, cb1, sb1, tT1, tB1)
        return nxt

    lax.fori_loop(0, _RPW // 2, pair_body, t0)
    pltpu.make_async_copy(emb_hbm.at[it0], ct0, st0).wait()
    pltpu.make_async_copy(emb_hbm.at[ib0], cb0, sb0).wait()
    pltpu.sync_copy(feat_v, out_hbm.at[pl.ds(base * 2 * _R, _RPW * 2 * _R)])


_sc_select = functools.partial(
    pl.kernel,
    out_type=jax.ShapeDtypeStruct((_NROW * 2 * _R,), jnp.float32),
    mesh=plsc.VectorSubcoreMesh(core_axis_name="c", subcore_axis_name="s",
                                num_cores=_NC, num_subcores=_NS),
    compiler_params=pltpu.CompilerParams(needs_layout_passes=False),
    scratch_types=[
        pltpu.VMEM((2 * _SROW,), jnp.float32),
        pltpu.VMEM((_L,), jnp.int32),
        pltpu.VMEM((_L,), jnp.int32),
        pltpu.VMEM((_L,), jnp.int32),
        pltpu.VMEM((_L,), jnp.int32),
        pltpu.VMEM((_L, _BLK), jnp.float32),
        pltpu.VMEM((_L, _BLK), jnp.float32),
        pltpu.VMEM((_L, _BLK), jnp.float32),
        pltpu.VMEM((_L, _BLK), jnp.float32),
        pltpu.VMEM((_L * _BLK + 2 * _L,), jnp.float32),
        pltpu.VMEM((_RPW * 2 * _R,), jnp.float32),
        pltpu.SemaphoreType.DMA,
        pltpu.SemaphoreType.DMA,
        pltpu.SemaphoreType.DMA,
        pltpu.SemaphoreType.DMA,
    ],
)(_sc_body)


def _mlp_body(f_ref, w1_ref, b1_ref, w2_ref, b2_ref, w3_ref, b3_ref, out_ref):
    def dot(a, b):
        return lax.dot_general(
            a, b, (((1,), (0,)), ((), ())),
            preferred_element_type=jnp.float32,
            precision=lax.Precision.HIGHEST,
        )

    h = dot(f_ref[...], w1_ref[...]) + b1_ref[...]
    h = 1.0 / (1.0 + jnp.exp(-h))
    h = dot(h, w2_ref[...]) + b2_ref[...]
    h = 1.0 / (1.0 + jnp.exp(-h))
    out_ref[...] = dot(h, w3_ref[...]) + b3_ref[...]


def kernel(x, Wc, bc, W1, b1, W2, b2, W3, b3):
    emb, bmax, bmin = pl.pallas_call(
        _emb_stats_body,
        grid=(_B,),
        in_specs=[
            pl.BlockSpec((1, _F, _N), lambda b: (b, 0, 0)),
            pl.BlockSpec((_J, _F), lambda b: (0, 0)),
            pl.BlockSpec((_J, 1), lambda b: (0, 0)),
        ],
        out_specs=[
            pl.BlockSpec((1, _J, _NBLK, _BLK), lambda b: (b, 0, 0, 0)),
            pl.BlockSpec((1, _J, _NBLK), lambda b: (b, 0, 0)),
            pl.BlockSpec((1, _J, _NBLK), lambda b: (b, 0, 0)),
        ],
        out_shape=[
            jax.ShapeDtypeStruct((_B, _J, _NBLK, _BLK), jnp.float32),
            jax.ShapeDtypeStruct((_B, _J, _NBLK), jnp.float32),
            jax.ShapeDtypeStruct((_B, _J, _NBLK), jnp.float32),
        ],
    )(x, Wc, bc.reshape(_J, 1))

    feats = _sc_select(emb.reshape(_NROW * _NBLK, _BLK),
                       bmax.reshape(-1), bmin.reshape(-1))

    flat = feats.reshape(_B, _J * 2 * _R)
    logits = pl.pallas_call(
        _mlp_body,
        out_shape=jax.ShapeDtypeStruct((_B, 2), jnp.float32),
    )(flat, W1, b1.reshape(1, -1), W2, b2.reshape(1, -1),
      W3, b3.reshape(1, -1))
    return logits


# speedup vs baseline: 110.2736x; 1.3861x over previous
import functools

import jax
import jax.numpy as jnp
from jax import lax
from jax.experimental import pallas as pl
from jax.experimental.pallas import tpu as pltpu
from jax.experimental.pallas import tpu_sc as plsc

_B, _F, _N = 32, 64, 32768
_J, _R = 32, 16
_BLK = 128
_NBLK = _N // _BLK
_L = 16
_NC, _NS = 2, 16
_NW = _NC * _NS
_NCH = 4
_BCH = _B // _NCH
_NROWC = _BCH * _J
_RPW = _NROWC // _NW
_SROW = _RPW * _NBLK


def _emb_stats_body(x_ref, wc_ref, bc_ref, emb_ref, bmax_ref, bmin_ref):
    xb = x_ref[0]
    emb = lax.dot_general(
        wc_ref[...], xb, (((1,), (0,)), ((), ())),
        preferred_element_type=jnp.float32,
    ) + bc_ref[...]
    r = emb.reshape(_J, _NBLK, _BLK)
    emb_ref[0] = r
    bmax_ref[0] = jnp.max(r, axis=2)
    bmin_ref[0] = jnp.min(r, axis=2)


def _sc_body(emb_hbm, bmax_hbm, bmin_hbm, out_hbm,
             stats_v, it0, ib0, it1, ib1, ct0, cb0, ct1, cb1,
             buf_v, feat_v, st0, sb0, st1, sb1):
    wid = lax.axis_index("s") * _NC + lax.axis_index("c")
    base = wid * _RPW
    pltpu.sync_copy(bmax_hbm.at[pl.ds(base * _NBLK, _SROW)],
                    stats_v.at[pl.ds(0, _SROW)])
    pltpu.sync_copy(bmin_hbm.at[pl.ds(base * _NBLK, _SROW)],
                    stats_v.at[pl.ds(_SROW, _SROW)])

    neg_inf = jnp.float32(-jnp.inf)
    lane = lax.iota(jnp.int32, _L)

    def select2(rloc):
        runvT = jnp.full((_L,), neg_inf)
        runiT = jnp.zeros((_L,), jnp.int32)
        runvB = jnp.full((_L,), neg_inf)
        runiB = jnp.zeros((_L,), jnp.int32)
        for c in range(_NBLK // _L):
            ids = lane + c * _L
            vT = stats_v[pl.ds(rloc * _NBLK + c * _L, _L)]
            vB = -stats_v[pl.ds(_SROW + rloc * _NBLK + c * _L, _L)]
            vdT, idT = plsc.sort_key_val(vT, ids, descending=True)
            vdB, idB = plsc.sort_key_val(vB, ids, descending=True)
            kT = runvT >= vdT
            kB = runvB >= vdB
            runvT, runiT = plsc.sort_key_val(jnp.where(kT, runvT, vdT),
                                             jnp.where(kT, runiT, idT))
            runvB, runiB = plsc.sort_key_val(jnp.where(kB, runvB, vdB),
                                             jnp.where(kB, runiB, idB))
        return runiT, jnp.min(runvT), runiB, jnp.min(runvB)

    def sel_fire(rloc, idxT, candT, semT, idxB, candB, semB):
        rl = jnp.where(rloc >= _RPW, 0, rloc)
        riT, tT, riB, tB = select2(rl)
        gbase = (base + rl) * _NBLK
        idxT[...] = riT + gbase
        pltpu.async_copy(emb_hbm.at[idxT], candT, semT)
        idxB[...] = riB + gbase
        pltpu.async_copy(emb_hbm.at[idxB], candB, semB)
        return tT, tB

    def process(cand, t, negate):
        def jbody(j, ptr):
            for k in range(_BLK // _L):
                cvec = cand[j, pl.ds(k * _L, _L)]
                if negate:
                    cvec = -cvec
                m = cvec >= t
                plsc.store_compressed(buf_v.at[pl.ds(ptr, _L)], cvec, mask=m)
                ptr = ptr + plsc.all_reduce_population_count(m)[0]
            return ptr
        ptr = lax.fori_loop(0, _L, jbody, jnp.int32(0))
        buf_v[pl.ds(ptr, _L)] = jnp.full((_L,), neg_inf)
        def fbody(i, rv):
            cd = plsc.sort_key_val(buf_v[pl.ds(i * _L, _L)], lane,
                                   descending=True)[0]
            return lax.sort(jnp.where(rv >= cd, rv, cd))
        nv = (ptr + _L - 1) // _L
        return lax.fori_loop(0, nv, fbody, jnp.full((_L,), neg_inf))

    def finish(rloc, idxT, candT, semT, idxB, candB, semB, tT, tB):
        pltpu.make_async_copy(emb_hbm.at[idxT], candT, semT).wait()
        pltpu.make_async_copy(emb_hbm.at[idxB], candB, semB).wait()
        top = process(candT, tT, False)
        feat_v[pl.ds(rloc * 2 * _R, _R)] = lax.rev(top, (0,))
        bot = process(candB, tB, True)
        feat_v[pl.ds(rloc * 2 * _R + _R, _R)] = -bot

    t0 = sel_fire(jnp.int32(0), it0, ct0, st0, ib0, cb0, sb0)

    def pair_body(i, carry):
        tT0, tB0 = carry
        r0 = 2 * i
        tT1, tB1 = sel_fire(r0 + 1, it1, ct1, st1, ib1, cb1, sb1)
        finish(r0, it0, ct0, st0, ib0, cb0, sb0, tT0, tB0)
        nxt = sel_fire(r0 + 2, it0, ct0, st0, ib0, cb0, sb0)
        finish(r0 + 1, it1, ct1, st1, ib1, cb1, sb1, tT1, tB1)
        return nxt

    lax.fori_loop(0, _RPW // 2, pair_body, t0)
    pltpu.make_async_copy(emb_hbm.at[it0], ct0, st0).wait()
    pltpu.make_async_copy(emb_hbm.at[ib0], cb0, sb0).wait()
    pltpu.sync_copy(feat_v, out_hbm.at[pl.ds(base * 2 * _R, _RPW * 2 * _R)])


_sc_select = functools.partial(
    pl.kernel,
    out_type=jax.ShapeDtypeStruct((_NROWC * 2 * _R,), jnp.float32),
    mesh=plsc.VectorSubcoreMesh(core_axis_name="c", subcore_axis_name="s",
                                num_cores=_NC, num_subcores=_NS),
    compiler_params=pltpu.CompilerParams(needs_layout_passes=False),
    scratch_types=[
        pltpu.VMEM((2 * _SROW,), jnp.float32),
        pltpu.VMEM((_L,), jnp.int32),
        pltpu.VMEM((_L,), jnp.int32),
        pltpu.VMEM((_L,), jnp.int32),
        pltpu.VMEM((_L,), jnp.int32),
        pltpu.VMEM((_L, _BLK), jnp.float32),
        pltpu.VMEM((_L, _BLK), jnp.float32),
        pltpu.VMEM((_L, _BLK), jnp.float32),
        pltpu.VMEM((_L, _BLK), jnp.float32),
        pltpu.VMEM((_L * _BLK + 2 * _L,), jnp.float32),
        pltpu.VMEM((_RPW * 2 * _R,), jnp.float32),
        pltpu.SemaphoreType.DMA,
        pltpu.SemaphoreType.DMA,
        pltpu.SemaphoreType.DMA,
        pltpu.SemaphoreType.DMA,
    ],
)(_sc_body)


def _mlp_body(f_ref, w1_ref, b1_ref, w2_ref, b2_ref, w3_ref, b3_ref, out_ref):
    def dot(a, b):
        return lax.dot_general(
            a, b, (((1,), (0,)), ((), ())),
            preferred_element_type=jnp.float32,
        )

    h = dot(f_ref[...], w1_ref[...]) + b1_ref[...]
    h = 1.0 / (1.0 + jnp.exp(-h))
    h = dot(h, w2_ref[...]) + b2_ref[...]
    h = 1.0 / (1.0 + jnp.exp(-h))
    out_ref[...] = dot(h, w3_ref[...]) + b3_ref[...]


def kernel(x, Wc, bc, W1, b1, W2, b2, W3, b3):
    bc2 = bc.reshape(_J, 1)
    feats = []
    for ch in range(_NCH):
        emb, bmax, bmin = pl.pallas_call(
            _emb_stats_body,
            grid=(_BCH,),
            in_specs=[
                pl.BlockSpec((1, _F, _N),
                             lambda b, ch=ch: (b + ch * _BCH, 0, 0)),
                pl.BlockSpec((_J, _F), lambda b: (0, 0)),
                pl.BlockSpec((_J, 1), lambda b: (0, 0)),
            ],
            out_specs=[
                pl.BlockSpec((1, _J, _NBLK, _BLK), lambda b: (b, 0, 0, 0)),
                pl.BlockSpec((1, _J, _NBLK), lambda b: (b, 0, 0)),
                pl.BlockSpec((1, _J, _NBLK), lambda b: (b, 0, 0)),
            ],
            out_shape=[
                jax.ShapeDtypeStruct((_BCH, _J, _NBLK, _BLK), jnp.float32),
                jax.ShapeDtypeStruct((_BCH, _J, _NBLK), jnp.float32),
                jax.ShapeDtypeStruct((_BCH, _J, _NBLK), jnp.float32),
            ],
        )(x, Wc, bc2)
        feats.append(_sc_select(emb.reshape(_NROWC * _NBLK, _BLK),
                                bmax.reshape(-1), bmin.reshape(-1)))

    flat = jnp.concatenate(feats).reshape(_B, _J * 2 * _R)
    logits = pl.pallas_call(
        _mlp_body,
        out_shape=jax.ShapeDtypeStruct((_B, 2), jnp.float32),
    )(flat, W1, b1.reshape(1, -1), W2, b2.reshape(1, -1),
      W3, b3.reshape(1, -1))
    return logits
